# baseline (speedup 1.0000x reference)
import functools

import jax
import jax.numpy as jnp
from jax import lax
from jax.experimental import pallas as pl
from jax.experimental.pallas import tpu as pltpu
from jax.experimental.pallas import tpu_sc as plsc

N_NODES = 10000
D = 128
N_EDGES = 320000

NC = 2
NS = 16
NW = NC * NS
L = 16

N_PAD = 10240
PAD_ROWS = N_PAD - N_NODES
EPW = N_PAD
E_PAD = NW * EPW
BLK = 1024
OBLK = 1000
CHUNK = 128
NCHUNK = EPW // CHUNK
SCHUNK = 64
BCHUNK = 128
NHALF = 4
HCHUNK = EPW // SCHUNK // NHALF
BHCHUNK = EPW // BCHUNK // NHALF
BGRP = BHCHUNK // 2
ROWS_PER_TILE = N_PAD // NS

_mesh = plsc.VectorSubcoreMesh(core_axis_name="c", subcore_axis_name="s")


def _zero_vec_loop(ref, n16):
    z = jnp.zeros((L,), jnp.float32)

    def body(i, c):
        ref[pl.ds(i * L, L)] = z
        return c

    lax.fori_loop(0, n16, body, 0)


@functools.partial(
    pl.kernel,
    out_type=jax.ShapeDtypeStruct((NC, N_PAD), jnp.float32),
    mesh=_mesh,
    scratch_types=[
        pltpu.VMEM((NCHUNK, CHUNK), jnp.int32),
        pltpu.VMEM((CHUNK,), jnp.float32),
        pltpu.VMEM((ROWS_PER_TILE,), jnp.float32),
        pltpu.VMEM_SHARED((N_PAD,), jnp.float32),
        pltpu.SemaphoreType.DMA,
    ],
)
def _deg_kernel(dst_hbm, deg_hbm, dst_v, ones_v, z_v, deg_sh, dsem):
    c = lax.axis_index("c")
    s = lax.axis_index("s")
    wid = s * NC + c

    pltpu.sync_copy(dst_hbm.at[wid], dst_v)

    one = jnp.ones((L,), jnp.float32)

    def fill_ones(i, cc):
        ones_v[pl.ds(i * L, L)] = one
        return cc

    lax.fori_loop(0, CHUNK // L, fill_ones, 0)
    _zero_vec_loop(z_v, ROWS_PER_TILE // L)
    pltpu.sync_copy(z_v, deg_sh.at[pl.ds(s * ROWS_PER_TILE, ROWS_PER_TILE)])
    plsc.subcore_barrier()

    def chunk(j, cc):
        for k in range(8):
            pltpu.async_copy(ones_v, deg_sh.at[dst_v.at[8 * j + k]],
                             dsem, add=True)
        for k in range(8):
            pltpu.make_async_copy(ones_v, deg_sh.at[dst_v.at[8 * j + k]],
                                  dsem).wait()
        return cc

    lax.fori_loop(0, NCHUNK // 8, chunk, 0)
    plsc.subcore_barrier()
    pltpu.sync_copy(
        deg_sh.at[pl.ds(s * ROWS_PER_TILE, ROWS_PER_TILE)],
        deg_hbm.at[c].at[pl.ds(s * ROWS_PER_TILE, ROWS_PER_TILE)],
    )


def _dense_body(x_ref, wc_ref, wf_ref, bias_ref, deg_ref, h2_ref, x0_ref):
    xb = x_ref[...]
    degb = deg_ref[:, 0] + deg_ref[:, 1]
    dis = jnp.where(degb > 0, lax.rsqrt(degb), 0.0)
    h = jnp.dot(xb, wc_ref[...], preferred_element_type=jnp.float32)
    h2_ref[...] = h * dis[:, None]
    x0_ref[...] = (
        jnp.dot(xb, wf_ref[...], preferred_element_type=jnp.float32)
        + bias_ref[0, :][None, :]
    )


@functools.partial(
    pl.kernel,
    out_type=jax.ShapeDtypeStruct((NC, N_PAD, D), jnp.float32),
    mesh=_mesh,
    scratch_types=[
        pltpu.VMEM((HCHUNK, SCHUNK), jnp.int32),
        pltpu.VMEM((BHCHUNK, BCHUNK), jnp.int32),
        pltpu.VMEM((BCHUNK, D), jnp.float32),
        pltpu.VMEM((BCHUNK, D), jnp.float32),
        pltpu.VMEM_SHARED((N_PAD, D), jnp.float32),
        pltpu.SemaphoreType.DMA,
        pltpu.SemaphoreType.DMA,
        pltpu.SemaphoreType.DMA,
        pltpu.SemaphoreType.DMA,
    ],
)
def _scatter_kernel(src_hbm, dst_hbm, h2_hbm, part_hbm,
                    src_v, dst_v, rows0, rows1, acc_sh,
                    semg0, semg1, sems0, sems1):
    c = lax.axis_index("c")
    s = lax.axis_index("s")
    wid = s * NC + c

    z = jnp.zeros((L,), jnp.float32)

    def zbody(i, cc):
        rows0[i // (D // L), pl.ds((i % (D // L)) * L, L)] = z
        return cc

    lax.fori_loop(0, BCHUNK * D // L, zbody, 0)
    for k in range(ROWS_PER_TILE // BCHUNK):
        pltpu.sync_copy(
            rows0, acc_sh.at[pl.ds(s * ROWS_PER_TILE + k * BCHUNK, BCHUNK)]
        )
    plsc.subcore_barrier()

    rows = (rows0, rows1)
    semg = (semg0, semg1)
    sems = (sems0, sems1)
    for stage in range(NHALF):
        pltpu.sync_copy(src_hbm.at[wid].at[stage], src_v)
        pltpu.sync_copy(dst_hbm.at[wid].at[stage], dst_v)
        for b in range(2):
            for hh in range(2):
                pltpu.async_copy(h2_hbm.at[src_v.at[2 * b + hh]],
                                 rows[b].at[pl.ds(hh * SCHUNK, SCHUNK)],
                                 semg[b])

        def grp(j, cc):
            for b in range(2):
                bc = 2 * j + b
                for hh in range(2):
                    pltpu.make_async_copy(
                        h2_hbm.at[src_v.at[2 * bc + hh]],
                        rows[b].at[pl.ds(hh * SCHUNK, SCHUNK)],
                        semg[b],
                    ).wait()
                pltpu.async_copy(rows[b], acc_sh.at[dst_v.at[bc]], sems[b],
                                 add=True)
            for b in range(2):
                bc = 2 * j + b
                pltpu.make_async_copy(
                    rows[b], acc_sh.at[dst_v.at[bc]], sems[b]
                ).wait()

                @pl.when(j < BGRP - 1)
                def _():
                    for hh in range(2):
                        pltpu.async_copy(
                            h2_hbm.at[src_v.at[2 * (bc + 2) + hh]],
                            rows[b].at[pl.ds(hh * SCHUNK, SCHUNK)],
                            semg[b],
                        )

            return cc

        lax.fori_loop(0, BGRP, grp, 0)
    plsc.subcore_barrier()
    pltpu.sync_copy(
        acc_sh.at[pl.ds(s * ROWS_PER_TILE, ROWS_PER_TILE)],
        part_hbm.at[c].at[pl.ds(s * ROWS_PER_TILE, ROWS_PER_TILE)],
    )


def _combine_body(p_ref, deg_ref, x0_ref, o_ref):
    degb = deg_ref[:, 0] + deg_ref[:, 1]
    dis = jnp.where(degb > 0, lax.rsqrt(degb), 0.0)
    o_ref[...] = (p_ref[0] + p_ref[1]) * dis[:, None] + x0_ref[...]


def kernel(x, edge_index, W_conv, b_conv, W_fc, b_fc):
    ei = edge_index.astype(jnp.int32)
    npad = E_PAD - N_EDGES
    pad_ids = N_NODES + (jnp.arange(npad, dtype=jnp.int32) % PAD_ROWS)
    src_flat = jnp.concatenate([ei[0], pad_ids])
    dst_flat = jnp.concatenate([ei[1], pad_ids])
    src = src_flat.reshape(NW, NHALF, HCHUNK, SCHUNK)
    dst = dst_flat.reshape(NW, NHALF, BHCHUNK, BCHUNK)
    dst_deg = dst_flat.reshape(NW, NCHUNK, CHUNK)
    x_pad = jnp.pad(x, ((0, PAD_ROWS), (0, 0)))
    bias = (b_conv + b_fc).reshape(1, D)

    deg = _deg_kernel(dst_deg)
    deg_t = deg.T

    h2, x0 = pl.pallas_call(
        _dense_body,
        out_shape=(
            jax.ShapeDtypeStruct((N_PAD, D), jnp.float32),
            jax.ShapeDtypeStruct((N_PAD, D), jnp.float32),
        ),
        grid=(N_PAD // BLK,),
        in_specs=[
            pl.BlockSpec((BLK, D), lambda i: (i, 0)),
            pl.BlockSpec((D, D), lambda i: (0, 0)),
            pl.BlockSpec((D, D), lambda i: (0, 0)),
            pl.BlockSpec((1, D), lambda i: (0, 0)),
            pl.BlockSpec((BLK, NC), lambda i: (i, 0)),
        ],
        out_specs=(
            pl.BlockSpec((BLK, D), lambda i: (i, 0)),
            pl.BlockSpec((BLK, D), lambda i: (i, 0)),
        ),
    )(x_pad, W_conv, W_fc, bias, deg_t)

    parts = _scatter_kernel(src, dst, h2)

    return pl.pallas_call(
        _combine_body,
        out_shape=jax.ShapeDtypeStruct((N_NODES, D), jnp.float32),
        grid=(N_NODES // OBLK,),
        in_specs=[
            pl.BlockSpec((NC, OBLK, D), lambda i: (0, i, 0)),
            pl.BlockSpec((OBLK, NC), lambda i: (i, 0)),
            pl.BlockSpec((OBLK, D), lambda i: (i, 0)),
        ],
        out_specs=pl.BlockSpec((OBLK, D), lambda i: (i, 0)),
    )(parts, deg_t, x0)


# speedup vs baseline: 1.1627x; 1.1627x over previous
import functools

import jax
import jax.numpy as jnp
from jax import lax
from jax.experimental import pallas as pl
from jax.experimental.pallas import tpu as pltpu
from jax.experimental.pallas import tpu_sc as plsc

N_NODES = 10000
D = 128
N_EDGES = 320000

NC = 2
NS = 16
NW = NC * NS
L = 16

N_PAD = 10240
PAD_ROWS = N_PAD - N_NODES
EPW = N_PAD
E_PAD = NW * EPW
BLK = 1024
OBLK = 1000
CHUNK = 128
NCHUNK = EPW // CHUNK
SCHUNK = 64
NHALF = 4
HCHUNK = EPW // SCHUNK // NHALF
NBUF = 4
HGRP = HCHUNK // NBUF
ROWS_PER_TILE = N_PAD // NS

_mesh = plsc.VectorSubcoreMesh(core_axis_name="c", subcore_axis_name="s")


def _zero_vec_loop(ref, n16):
    z = jnp.zeros((L,), jnp.float32)

    def body(i, c):
        ref[pl.ds(i * L, L)] = z
        return c

    lax.fori_loop(0, n16, body, 0)


@functools.partial(
    pl.kernel,
    out_type=jax.ShapeDtypeStruct((NC, N_PAD), jnp.float32),
    mesh=_mesh,
    scratch_types=[
        pltpu.VMEM((NCHUNK, CHUNK), jnp.int32),
        pltpu.VMEM((CHUNK,), jnp.float32),
        pltpu.VMEM((ROWS_PER_TILE,), jnp.float32),
        pltpu.VMEM_SHARED((N_PAD,), jnp.float32),
        pltpu.SemaphoreType.DMA,
    ],
)
def _deg_kernel(dst_hbm, deg_hbm, dst_v, ones_v, z_v, deg_sh, dsem):
    c = lax.axis_index("c")
    s = lax.axis_index("s")
    wid = s * NC + c

    pltpu.sync_copy(dst_hbm.at[wid], dst_v)

    one = jnp.ones((L,), jnp.float32)

    def fill_ones(i, cc):
        ones_v[pl.ds(i * L, L)] = one
        return cc

    lax.fori_loop(0, CHUNK // L, fill_ones, 0)
    _zero_vec_loop(z_v, ROWS_PER_TILE // L)
    pltpu.sync_copy(z_v, deg_sh.at[pl.ds(s * ROWS_PER_TILE, ROWS_PER_TILE)])
    plsc.subcore_barrier()

    def chunk(j, cc):
        for k in range(8):
            pltpu.async_copy(ones_v, deg_sh.at[dst_v.at[8 * j + k]],
                             dsem, add=True)
        for k in range(8):
            pltpu.make_async_copy(ones_v, deg_sh.at[dst_v.at[8 * j + k]],
                                  dsem).wait()
        return cc

    lax.fori_loop(0, NCHUNK // 8, chunk, 0)
    plsc.subcore_barrier()
    pltpu.sync_copy(
        deg_sh.at[pl.ds(s * ROWS_PER_TILE, ROWS_PER_TILE)],
        deg_hbm.at[c].at[pl.ds(s * ROWS_PER_TILE, ROWS_PER_TILE)],
    )


def _dense_body(x_ref, wc_ref, wf_ref, bias_ref, deg_ref, h2_ref, x0_ref):
    xb = x_ref[...]
    degb = deg_ref[:, 0] + deg_ref[:, 1]
    dis = jnp.where(degb > 0, lax.rsqrt(degb), 0.0)
    h = jnp.dot(xb, wc_ref[...], preferred_element_type=jnp.float32)
    h2_ref[...] = h * dis[:, None]
    x0_ref[...] = (
        jnp.dot(xb, wf_ref[...], preferred_element_type=jnp.float32)
        + bias_ref[0, :][None, :]
    )


@functools.partial(
    pl.kernel,
    out_type=jax.ShapeDtypeStruct((NC, N_PAD, D), jnp.float32),
    mesh=_mesh,
    scratch_types=[
        pltpu.VMEM((HCHUNK, SCHUNK), jnp.int32),
        pltpu.VMEM((HCHUNK, SCHUNK), jnp.int32),
        pltpu.VMEM((SCHUNK, D), jnp.float32),
        pltpu.VMEM((SCHUNK, D), jnp.float32),
        pltpu.VMEM((SCHUNK, D), jnp.float32),
        pltpu.VMEM((SCHUNK, D), jnp.float32),
        pltpu.VMEM_SHARED((N_PAD, D), jnp.float32),
        pltpu.SemaphoreType.DMA,
        pltpu.SemaphoreType.DMA,
        pltpu.SemaphoreType.DMA,
        pltpu.SemaphoreType.DMA,
        pltpu.SemaphoreType.DMA,
        pltpu.SemaphoreType.DMA,
        pltpu.SemaphoreType.DMA,
        pltpu.SemaphoreType.DMA,
    ],
)
def _scatter_kernel(src_hbm, dst_hbm, h2_hbm, part_hbm,
                    src_v, dst_v, rows0, rows1, rows2, rows3, acc_sh,
                    semg0, semg1, semg2, semg3,
                    sems0, sems1, sems2, sems3):
    c = lax.axis_index("c")
    s = lax.axis_index("s")
    wid = s * NC + c

    z = jnp.zeros((L,), jnp.float32)

    def zbody(i, cc):
        rows0[i // (D // L), pl.ds((i % (D // L)) * L, L)] = z
        return cc

    lax.fori_loop(0, SCHUNK * D // L, zbody, 0)
    for k in range(ROWS_PER_TILE // SCHUNK):
        pltpu.sync_copy(
            rows0, acc_sh.at[pl.ds(s * ROWS_PER_TILE + k * SCHUNK, SCHUNK)]
        )
    plsc.subcore_barrier()

    rows = (rows0, rows1, rows2, rows3)
    semg = (semg0, semg1, semg2, semg3)
    sems = (sems0, sems1, sems2, sems3)
    for stage in range(NHALF):
        pltpu.sync_copy(src_hbm.at[wid].at[stage], src_v)
        pltpu.sync_copy(dst_hbm.at[wid].at[stage], dst_v)
        for k in range(NBUF):
            pltpu.async_copy(h2_hbm.at[src_v.at[k]], rows[k], semg[k])

        def grp(j, cc):
            for k in range(NBUF):
                c = NBUF * j + k
                pltpu.make_async_copy(
                    h2_hbm.at[src_v.at[c]], rows[k], semg[k]
                ).wait()
                pltpu.async_copy(rows[k], acc_sh.at[dst_v.at[c]], sems[k],
                                 add=True)
            for k in range(NBUF):
                c = NBUF * j + k
                pltpu.make_async_copy(
                    rows[k], acc_sh.at[dst_v.at[c]], sems[k]
                ).wait()

                @pl.when(j < HGRP - 1)
                def _():
                    pltpu.async_copy(h2_hbm.at[src_v.at[c + NBUF]],
                                     rows[k], semg[k])

            return cc

        lax.fori_loop(0, HGRP, grp, 0)
    plsc.subcore_barrier()
    pltpu.sync_copy(
        acc_sh.at[pl.ds(s * ROWS_PER_TILE, ROWS_PER_TILE)],
        part_hbm.at[c].at[pl.ds(s * ROWS_PER_TILE, ROWS_PER_TILE)],
    )


def _combine_body(p_ref, deg_ref, x0_ref, o_ref):
    degb = deg_ref[:, 0] + deg_ref[:, 1]
    dis = jnp.where(degb > 0, lax.rsqrt(degb), 0.0)
    o_ref[...] = (p_ref[0] + p_ref[1]) * dis[:, None] + x0_ref[...]


def kernel(x, edge_index, W_conv, b_conv, W_fc, b_fc):
    ei = edge_index.astype(jnp.int32)
    npad = E_PAD - N_EDGES
    pad_ids = N_NODES + (jnp.arange(npad, dtype=jnp.int32) % PAD_ROWS)
    src_flat = jnp.concatenate([ei[0], pad_ids])
    dst_flat = jnp.concatenate([ei[1], pad_ids])
    src = src_flat.reshape(NW, NHALF, HCHUNK, SCHUNK)
    dst = dst_flat.reshape(NW, NHALF, HCHUNK, SCHUNK)
    dst_deg = dst_flat.reshape(NW, NCHUNK, CHUNK)
    x_pad = jnp.pad(x, ((0, PAD_ROWS), (0, 0)))
    bias = (b_conv + b_fc).reshape(1, D)

    deg = _deg_kernel(dst_deg)
    deg_t = deg.T

    h2, x0 = pl.pallas_call(
        _dense_body,
        out_shape=(
            jax.ShapeDtypeStruct((N_PAD, D), jnp.float32),
            jax.ShapeDtypeStruct((N_PAD, D), jnp.float32),
        ),
        grid=(N_PAD // BLK,),
        in_specs=[
            pl.BlockSpec((BLK, D), lambda i: (i, 0)),
            pl.BlockSpec((D, D), lambda i: (0, 0)),
            pl.BlockSpec((D, D), lambda i: (0, 0)),
            pl.BlockSpec((1, D), lambda i: (0, 0)),
            pl.BlockSpec((BLK, NC), lambda i: (i, 0)),
        ],
        out_specs=(
            pl.BlockSpec((BLK, D), lambda i: (i, 0)),
            pl.BlockSpec((BLK, D), lambda i: (i, 0)),
        ),
    )(x_pad, W_conv, W_fc, bias, deg_t)

    parts = _scatter_kernel(src, dst, h2)

    return pl.pallas_call(
        _combine_body,
        out_shape=jax.ShapeDtypeStruct((N_NODES, D), jnp.float32),
        grid=(N_NODES // OBLK,),
        in_specs=[
            pl.BlockSpec((NC, OBLK, D), lambda i: (0, i, 0)),
            pl.BlockSpec((OBLK, NC), lambda i: (i, 0)),
            pl.BlockSpec((OBLK, D), lambda i: (i, 0)),
        ],
        out_specs=pl.BlockSpec((OBLK, D), lambda i: (i, 0)),
    )(parts, deg_t, x0)
